# baseline (speedup 1.0000x reference)
import functools

import jax
import jax.numpy as jnp
from jax.experimental import pallas as pl
from jax.experimental.pallas import tpu as pltpu

_BS, _A, _H, _W, _NC = 64, 3, 52, 52, 80
_ATTRS = 5 + _NC


def _softplus(z):
    return jnp.maximum(z, 0.0) + jnp.log(1.0 + jnp.exp(-jnp.abs(z)))


def _loss_kernel(pred0_ref, pred1_ref, pred2_ref, tcls0_ref, tcls1_ref,
                 tcls2_ref, mask_ref, noobj_ref, tx_ref, ty_ref, tw_ref,
                 th_ref, bsx_ref, bsy_ref, out_ref, acc_ref):
    b = pl.program_id(0)

    @pl.when(b == 0)
    def _init():
        acc_ref[...] = jnp.zeros_like(acc_ref)

    acc = acc_ref[...]
    pred_refs = (pred0_ref, pred1_ref, pred2_ref)
    tcls_refs = (tcls0_ref, tcls1_ref, tcls2_ref)
    for a in range(_A):
        m = mask_ref[0, a]
        nm = noobj_ref[0, a]
        t_x = tx_ref[0, a]
        t_y = ty_ref[0, a]
        t_w = tw_ref[0, a]
        t_h = th_ref[0, a]
        sx = bsx_ref[0, a]
        sy = bsy_ref[0, a]

        pa = pred_refs[a]
        zx = pa[0, 0]
        zy = pa[0, 1]
        zw = pa[0, 2]
        zh = pa[0, 3]
        zc = pa[0, 4]
        zcls = pa[0, 5:]

        dw = zw - t_w
        dh = zh - t_h
        box = (_softplus(zx) - t_x * zx) + (_softplus(zy) - t_y * zy) \
            + dw * dw + dh * dh
        plane = box * ((2.0 - sx * sy) * m)
        plane += (_softplus(zc) - m * zc) * (m + nm)

        tcls_t = jnp.transpose(tcls_refs[a][0, 0], (2, 0, 1))
        cls_term = jnp.sum(_softplus(zcls) - tcls_t * zcls, axis=0)
        plane += cls_term * m
        acc += plane
    acc_ref[...] = acc

    @pl.when(b == _BS - 1)
    def _finish():
        out_ref[0, 0] = jnp.sum(acc_ref[...]) * (1.0 / _BS)


@functools.partial(jax.jit, static_argnames=("interpret",))
def kernel(pred, mask, noobj_mask, tx, ty, tw, th, tcls,
           box_loss_scale_x, box_loss_scale_y, interpret=False):
    plane = pl.BlockSpec((1, _A, _H, _W), lambda b: (b, 0, 0, 0))

    def pred_spec(a):
        return pl.BlockSpec((1, _ATTRS, _H, _W), lambda b, a=a: (b, a, 0, 0))

    def tcls_spec(a):
        return pl.BlockSpec((1, 1, _H, _W, _NC),
                            lambda b, a=a: (b, a, 0, 0, 0))

    out = pl.pallas_call(
        _loss_kernel,
        grid=(_BS,),
        in_specs=[
            pred_spec(0), pred_spec(1), pred_spec(2),
            tcls_spec(0), tcls_spec(1), tcls_spec(2),
            plane, plane, plane, plane, plane, plane, plane, plane,
        ],
        out_specs=pl.BlockSpec(
            (1, 1), lambda b: (0, 0), memory_space=pltpu.SMEM),
        out_shape=jax.ShapeDtypeStruct((1, 1), jnp.float32),
        scratch_shapes=[pltpu.VMEM((_H, _W), jnp.float32)],
        interpret=interpret,
    )(pred, pred, pred, tcls, tcls, tcls, mask, noobj_mask, tx, ty,
      tw, th, box_loss_scale_x, box_loss_scale_y)
    return out[0, 0]
